# baseline (speedup 1.0000x reference)
import functools

import jax
import jax.numpy as jnp
from jax import lax
from jax.experimental import pallas as pl
from jax.experimental.pallas import tpu as pltpu
from jax.experimental.pallas import tpu_sc as plsc

NUM_NODES = 1000000
EMBED_DIM = 32
BATCH = 16384
HIST = 50

_NC, _NS = 2, 16
_NW = _NC * _NS
_CHUNK = 128
_BB_PER_W = 4
_B_PER_W = _BB_PER_W * _CHUNK
_NCHUNK = _BB_PER_W * HIST
_FG = EMBED_DIM // 8


def _gather_body(idxt_hbm, table_hbm, p_hbm, idx_v, cbuf0, cbuf1,
                 tbuf0, tbuf1, gsem0, gsem1, wsem0, wsem1):
    wid = lax.axis_index("s") * _NC + lax.axis_index("c")

    pltpu.sync_copy(idxt_hbm.at[:, pl.ds(wid * _B_PER_W, _B_PER_W)], idx_v)

    base_iota = lax.iota(jnp.int32, 16)
    hi_iota = base_iota + 16

    def fire(c, cbuf, gsem):
        h = lax.rem(c, HIST)
        lbb = lax.div(c, HIST)
        pltpu.async_copy(
            table_hbm.at[idx_v.at[h, pl.ds(lbb * _CHUNK, _CHUNK)]],
            cbuf, gsem)

    def drain_gather(cbuf, gsem):
        pltpu.make_async_copy(table_hbm.at[pl.ds(0, _CHUNK)], cbuf, gsem).wait()

    def transpose(cbuf, tbuf):
        def body(i, _):
            bi0 = i * 8
            for r in range(8):
                bivec = jnp.zeros((16,), jnp.int32) + (bi0 + r)
                lo = cbuf[bi0 + r, pl.ds(0, 16)]
                hi = cbuf[bi0 + r, pl.ds(16, 16)]
                plsc.store_scatter(tbuf, [base_iota, bivec], lo)
                plsc.store_scatter(tbuf, [hi_iota, bivec], hi)
            return 0
        lax.fori_loop(0, _CHUNK // 8, body, 0)

    def start_write(c, tbuf, wsem):
        h = lax.rem(c, HIST)
        wbb = wid * _BB_PER_W + lax.div(c, HIST)
        for fg in range(_FG):
            pltpu.async_copy(tbuf.at[pl.ds(fg * 8, 8), pl.ds(0, _CHUNK)],
                             p_hbm.at[h, fg, wbb], wsem)

    def wait_write(tbuf, wsem):
        for fg in range(_FG):
            pltpu.make_async_copy(p_hbm.at[0, 0, 0],
                                  tbuf.at[pl.ds(0, 8), pl.ds(0, _CHUNK)],
                                  wsem).wait()

    fire(0, cbuf0, gsem0)
    fire(1, cbuf1, gsem1)

    def chunk_step(c, cbuf, tbuf, gsem, wsem, t):
        drain_gather(cbuf, gsem)

        @pl.when(t > 0)
        def _():
            wait_write(tbuf, wsem)
        transpose(cbuf, tbuf)

        @pl.when(c + 2 < _NCHUNK)
        def _():
            fire(c + 2, cbuf, gsem)
        start_write(c, tbuf, wsem)

    def outer(t, _):
        chunk_step(2 * t, cbuf0, tbuf0, gsem0, wsem0, t)
        chunk_step(2 * t + 1, cbuf1, tbuf1, gsem1, wsem1, t)
        return 0

    lax.fori_loop(0, _NCHUNK // 2, outer, 0)
    wait_write(tbuf0, wsem0)
    wait_write(tbuf1, wsem1)


_sc_gather = pl.kernel(
    _gather_body,
    mesh=plsc.VectorSubcoreMesh(core_axis_name="c", subcore_axis_name="s"),
    out_type=jax.ShapeDtypeStruct((HIST, _FG, BATCH // _CHUNK, 8, _CHUNK),
                                  jnp.float32),
    scratch_types=[
        pltpu.VMEM((HIST, _B_PER_W), jnp.int32),
        pltpu.VMEM((_CHUNK, EMBED_DIM), jnp.float32),
        pltpu.VMEM((_CHUNK, EMBED_DIM), jnp.float32),
        pltpu.VMEM((EMBED_DIM, 133), jnp.float32),
        pltpu.VMEM((EMBED_DIM, 133), jnp.float32),
        pltpu.SemaphoreType.DMA,
        pltpu.SemaphoreType.DMA,
        pltpu.SemaphoreType.DMA,
        pltpu.SemaphoreType.DMA,
    ],
    compiler_params=pltpu.CompilerParams(use_tc_tiling_on_sc=False,
                                         needs_layout_passes=False),
)


_TN = 512
_TGRID = -(-NUM_NODES // _TN)


def _tc_untile_body(in_ref, out_ref):
    x = in_ref[...]
    x3 = x.reshape(_TN // 8, 8, EMBED_DIM)
    rows = jnp.broadcast_to((lax.iota(jnp.int32, 2) * 4)[None, :, None],
                            (_TN // 8, 2, EMBED_DIM))
    parts = [jnp.take_along_axis(x3, rows + q, axis=1) for q in range(4)]
    out_ref[...] = jnp.concatenate(parts, axis=2).reshape(_TN // 4, 128)


_tc_untile = pl.pallas_call(
    _tc_untile_body,
    grid=(_TGRID,),
    in_specs=[pl.BlockSpec((_TN, EMBED_DIM), lambda k: (k, 0))],
    out_specs=pl.BlockSpec((_TN // 4, 128), lambda k: (k, 0)),
    out_shape=jax.ShapeDtypeStruct((NUM_NODES * EMBED_DIM // 128, 128),
                                   jnp.float32),
)


def kernel(indices, embeddings):
    table_lin = _tc_untile(embeddings).reshape(NUM_NODES, EMBED_DIM)
    p = _sc_gather(indices.T, table_lin)
    return p.transpose(2, 4, 0, 1, 3).reshape(BATCH, HIST, EMBED_DIM)


# speedup vs baseline: 2.7377x; 2.7377x over previous
import functools

import jax
import jax.numpy as jnp
from jax import lax
from jax.experimental import pallas as pl
from jax.experimental.pallas import tpu as pltpu
from jax.experimental.pallas import tpu_sc as plsc

NUM_NODES = 1000000
EMBED_DIM = 32
BATCH = 16384
HIST = 50

_NC, _NS = 2, 16
_NW = _NC * _NS
_CHUNK = 128
_BB_PER_W = 4
_B_PER_W = _BB_PER_W * _CHUNK
_NCHUNK = _BB_PER_W * HIST
_FG = EMBED_DIM // 8


def _gather_body(idxt_hbm, table_hbm, p_hbm, idx_v, cbuf0, cbuf1,
                 tbuf0, tbuf1, gsem0, gsem1, wsem0, wsem1):
    wid = lax.axis_index("s") * _NC + lax.axis_index("c")

    pltpu.sync_copy(idxt_hbm.at[:, pl.ds(wid * _B_PER_W, _B_PER_W)], idx_v)

    base_iota = lax.iota(jnp.int32, 16)
    hi_iota = base_iota + 16

    def fire(c, cbuf, gsem):
        h = lax.rem(c, HIST)
        lbb = lax.div(c, HIST)
        pltpu.async_copy(
            table_hbm.at[idx_v.at[h, pl.ds(lbb * _CHUNK, _CHUNK)]],
            cbuf, gsem)

    def drain_gather(cbuf, gsem):
        pltpu.make_async_copy(table_hbm.at[pl.ds(0, _CHUNK)], cbuf, gsem).wait()

    def transpose(cbuf, tbuf):
        def body(i, _):
            bi0 = i * 8
            for r in range(8):
                bivec = jnp.zeros((16,), jnp.int32) + (bi0 + r)
                lo = cbuf[bi0 + r, pl.ds(0, 16)]
                hi = cbuf[bi0 + r, pl.ds(16, 16)]
                plsc.store_scatter(tbuf, [base_iota, bivec], lo)
                plsc.store_scatter(tbuf, [hi_iota, bivec], hi)
            return 0
        lax.fori_loop(0, _CHUNK // 8, body, 0)

    def start_write(c, tbuf, wsem):
        h = lax.rem(c, HIST)
        wbb = wid * _BB_PER_W + lax.div(c, HIST)
        for fg in range(_FG):
            pltpu.async_copy(tbuf.at[pl.ds(fg * 8, 8), pl.ds(0, _CHUNK)],
                             p_hbm.at[h, fg, wbb], wsem)

    def wait_write(tbuf, wsem):
        for fg in range(_FG):
            pltpu.make_async_copy(p_hbm.at[0, 0, 0],
                                  tbuf.at[pl.ds(0, 8), pl.ds(0, _CHUNK)],
                                  wsem).wait()

    fire(0, cbuf0, gsem0)
    fire(1, cbuf1, gsem1)

    def chunk_step(c, cbuf, tbuf, gsem, wsem, t):
        drain_gather(cbuf, gsem)

        @pl.when(t > 0)
        def _():
            wait_write(tbuf, wsem)
        transpose(cbuf, tbuf)

        @pl.when(c + 2 < _NCHUNK)
        def _():
            fire(c + 2, cbuf, gsem)
        start_write(c, tbuf, wsem)

    def outer(t, _):
        chunk_step(2 * t, cbuf0, tbuf0, gsem0, wsem0, t)
        chunk_step(2 * t + 1, cbuf1, tbuf1, gsem1, wsem1, t)
        return 0

    lax.fori_loop(0, _NCHUNK // 2, outer, 0)
    wait_write(tbuf0, wsem0)
    wait_write(tbuf1, wsem1)


_sc_gather = pl.kernel(
    _gather_body,
    mesh=plsc.VectorSubcoreMesh(core_axis_name="c", subcore_axis_name="s"),
    out_type=jax.ShapeDtypeStruct((HIST, _FG, BATCH // _CHUNK, 8, _CHUNK),
                                  jnp.float32),
    scratch_types=[
        pltpu.VMEM((HIST, _B_PER_W), jnp.int32),
        pltpu.VMEM((_CHUNK, EMBED_DIM), jnp.float32),
        pltpu.VMEM((_CHUNK, EMBED_DIM), jnp.float32),
        pltpu.VMEM((EMBED_DIM, 133), jnp.float32),
        pltpu.VMEM((EMBED_DIM, 133), jnp.float32),
        pltpu.SemaphoreType.DMA,
        pltpu.SemaphoreType.DMA,
        pltpu.SemaphoreType.DMA,
        pltpu.SemaphoreType.DMA,
    ],
    compiler_params=pltpu.CompilerParams(use_tc_tiling_on_sc=False,
                                         needs_layout_passes=False),
)


def kernel(indices, embeddings):
    p = _sc_gather(indices.T, embeddings)
    return p.transpose(2, 4, 0, 1, 3).reshape(BATCH, HIST, EMBED_DIM)
